# baseline (speedup 1.0000x reference)
import functools

import jax
import jax.numpy as jnp
from jax import lax
from jax.experimental import pallas as pl
from jax.experimental.pallas import tpu as pltpu
from jax.experimental.pallas import tpu_sc as plsc

N = 10000
E = 320000
D = 128
K = 10

NC = 2
NS = 16
DH = D // NC

C = 125
ES = E // NS
NCH = ES // C

RT = N // NS
NZ = RT // C
DT = 624
DREM = N - NS * DT
RPAD = 640

_mesh = plsc.VectorSubcoreMesh(core_axis_name="c", subcore_axis_name="s")
_sc_params = pltpu.CompilerParams(use_tc_tiling_on_sc=False)


def _newton_rsqrt(x):
    xi = lax.bitcast_convert_type(x, jnp.int32)
    yi = jnp.int32(0x5F3759DF) - (xi >> 1)
    y = lax.bitcast_convert_type(yi, jnp.float32)
    for _ in range(3):
        y = y * (1.5 - 0.5 * x * y * y)
    return y


@functools.partial(
    pl.kernel,
    out_type=[
        jax.ShapeDtypeStruct((K + 1, NC, N, DH), jnp.float32),
        jax.ShapeDtypeStruct((N, 16), jnp.float32),
    ],
    mesh=_mesh,
    scratch_types=[
        pltpu.VMEM((NCH, C), jnp.int32),
        pltpu.VMEM((NCH, C), jnp.int32),
        pltpu.VMEM((C, DH), jnp.float32),
        pltpu.VMEM((C, DH), jnp.float32),
        pltpu.VMEM((C, DH), jnp.float32),
        pltpu.VMEM((C, DH), jnp.float32),
        pltpu.VMEM((C, 16), jnp.float32),
        pltpu.SMEM((RPAD,), jnp.float32),
        pltpu.SMEM((RPAD,), jnp.float32),
        pltpu.VMEM_SHARED((N, DH), jnp.float32),
        pltpu.VMEM_SHARED((N, 16), jnp.float32),
        pltpu.SemaphoreType.DMA,
        pltpu.SemaphoreType.DMA,
        pltpu.SemaphoreType.DMA,
        pltpu.SemaphoreType.DMA,
        pltpu.SemaphoreType.DMA,
        pltpu.SemaphoreType.DMA,
        pltpu.SemaphoreType.DMA,
        pltpu.SemaphoreType.DMA,
    ],
    compiler_params=_sc_params,
)
def _multihop_kernel(fsplit_hbm, src_hbm, dst_hbm, gall_hbm, degs_hbm,
                     srcv, dstv, b0, b1, b2, b3, small, invds, normds,
                     acc, dacc,
                     sg0, sg1, sg2, sg3, ss0, ss1, ss2, ss3):
    c = lax.axis_index("c")
    s = lax.axis_index("s")
    bufs = (b0, b1, b2, b3)
    gsems = (sg0, sg1, sg2, sg3)
    ssems = (ss0, ss1, ss2, ss3)

    pltpu.sync_copy(src_hbm.at[s], srcv)
    pltpu.sync_copy(dst_hbm.at[s], dstv)

    def _zsmall(r, _):
        small[r, :] = jnp.zeros((16,), jnp.float32)
        return 0
    lax.fori_loop(0, C, _zsmall, 0)
    for q in range(NZ):
        pltpu.sync_copy(small, dacc.at[pl.ds(s * RT + q * C, C)])
    plsc.subcore_barrier()

    def _osmall(r, _):
        small[r, :] = jnp.ones((16,), jnp.float32)
        return 0
    lax.fori_loop(0, C, _osmall, 0)

    def _dchunk(j, _):
        pltpu.sync_copy(small, dacc.at[dstv.at[j]], add=True)
        return 0
    lax.fori_loop(0, NCH, _dchunk, 0)
    plsc.subcore_barrier()

    @pl.when(c == 0)
    def _():
        pltpu.sync_copy(dacc.at[pl.ds(s * DT, DT)],
                        degs_hbm.at[pl.ds(s * DT, DT)])

        @pl.when(s == NS - 1)
        def _():
            pltpu.sync_copy(dacc.at[pl.ds(NS * DT, DREM)],
                            degs_hbm.at[pl.ds(NS * DT, DREM)])

    for q in range(NZ):
        base = s * RT + q * C
        pltpu.sync_copy(dacc.at[pl.ds(base, C)], small)

        def _scal(r, _):
            dm = jnp.maximum(small[r, :], 1.0)
            iv = 1.0 / dm
            nr = _newton_rsqrt(dm)
            invds[q * C + r] = iv[0]
            normds[q * C + r] = nr[0]
            return 0
        lax.fori_loop(0, C, _scal, 0)

        pltpu.sync_copy(fsplit_hbm.at[c, pl.ds(base, C)], b0)

        def _g0row(r, _):
            nr = normds[q * C + r]
            for d4 in range(DH // 16):
                sl = pl.ds(d4 * 16, 16)
                b0[r, sl] = b0[r, sl] * nr
            return 0
        lax.fori_loop(0, C, _g0row, 0)
        pltpu.sync_copy(b0, gall_hbm.at[0, c, pl.ds(base, C)])
    plsc.subcore_barrier()

    def run_hop(k):
        gsrc = gall_hbm.at[k, c]

        def _zrow(r, _):
            for d4 in range(DH // 16):
                b3[r, pl.ds(d4 * 16, 16)] = jnp.zeros((16,), jnp.float32)
            return 0
        lax.fori_loop(0, C, _zrow, 0)
        for q in range(NZ):
            pltpu.sync_copy(b3, acc.at[pl.ds(s * RT + q * C, C)])
        plsc.subcore_barrier()

        def gstart(j, u):
            pltpu.async_copy(gsrc.at[srcv.at[j]], bufs[u], gsems[u])

        def gwait(j, u):
            pltpu.make_async_copy(gsrc.at[srcv.at[j]], bufs[u],
                                  gsems[u]).wait()

        def sstart(j, u):
            pltpu.async_copy(bufs[u], acc.at[dstv.at[j]], ssems[u],
                             add=True)

        def swait(j, u):
            pltpu.make_async_copy(bufs[u], acc.at[dstv.at[j]],
                                  ssems[u]).wait()

        gstart(0, 0)
        gstart(1, 1)

        def _group(i, _):
            for u in range(4):
                j = 4 * i + u
                gwait(j, u)
                sstart(j, u)
                if u >= 2:
                    swait(j - 2, (u - 2) % 4)
                else:
                    @pl.when(j >= 2)
                    def _():
                        swait(j - 2, (u - 2) % 4)
                if u < 2:
                    gstart(j + 2, (u + 2) % 4)
                else:
                    @pl.when(j + 2 < NCH)
                    def _():
                        gstart(j + 2, (u + 2) % 4)
            return 0

        lax.fori_loop(0, NCH // 4, _group, 0)
        swait(NCH - 2, (NCH - 2) % 4)
        swait(NCH - 1, (NCH - 1) % 4)
        plsc.subcore_barrier()

        for q in range(NZ):
            base = s * RT + q * C
            pltpu.sync_copy(acc.at[pl.ds(base, C)], b0)

            def _srow(r, _):
                iv = invds[q * C + r]
                for d4 in range(DH // 16):
                    sl = pl.ds(d4 * 16, 16)
                    b0[r, sl] = b0[r, sl] * iv
                return 0
            lax.fori_loop(0, C, _srow, 0)
            pltpu.sync_copy(b0, gall_hbm.at[k + 1, c, pl.ds(base, C)])
        plsc.subcore_barrier()

    def _hop(k, _):
        run_hop(k)
        return 0
    lax.fori_loop(0, K, _hop, 0)


_NB = 2000
_GRID = N // _NB


def _tc_final_body(gall_ref, degs_ref, s_ref, out_ref):
    k = pl.program_id(1)
    sqd = jnp.sqrt(jnp.maximum(degs_ref[:, 0:1], 1.0))
    g = jnp.concatenate([gall_ref[0, 0], gall_ref[0, 1]], axis=-1)
    h = g * sqd
    t = jnp.dot(h, s_ref[...], preferred_element_type=jnp.float32)
    term = jax.nn.sigmoid(t) * h

    @pl.when(k == 0)
    def _():
        out_ref[...] = term

    @pl.when(k > 0)
    def _():
        out_ref[...] += term


_tc_final = pl.pallas_call(
    _tc_final_body,
    grid=(_GRID, K + 1),
    in_specs=[
        pl.BlockSpec((1, NC, _NB, DH), lambda i, k: (k, 0, i, 0)),
        pl.BlockSpec((_NB, 16), lambda i, k: (i, 0)),
        pl.BlockSpec((D, 1), lambda i, k: (0, 0)),
    ],
    out_specs=pl.BlockSpec((_NB, D), lambda i, k: (i, 0)),
    out_shape=jax.ShapeDtypeStruct((N, D), jnp.float32),
)


def kernel(features, edge_index, s):
    src = edge_index[0].reshape(NS, NCH, C)
    dst = edge_index[1].reshape(NS, NCH, C)
    fsplit = jnp.stack([features[:, :DH], features[:, DH:]])
    gall, degs = _multihop_kernel(fsplit, src, dst)
    return _tc_final(gall, degs, s)


# speedup vs baseline: 1.0142x; 1.0142x over previous
import functools

import jax
import jax.numpy as jnp
from jax import lax
from jax.experimental import pallas as pl
from jax.experimental.pallas import tpu as pltpu
from jax.experimental.pallas import tpu_sc as plsc

N = 10000
E = 320000
D = 128
K = 10

NC = 2
NS = 16
DH = D // NC

C = 125
ES = E // NS
NCH = ES // C

RT = N // NS
NZ = RT // C
DT = 624
DREM = N - NS * DT
RPAD = 640

_mesh = plsc.VectorSubcoreMesh(core_axis_name="c", subcore_axis_name="s")
_sc_params = pltpu.CompilerParams(use_tc_tiling_on_sc=False)


def _newton_rsqrt(x):
    xi = lax.bitcast_convert_type(x, jnp.int32)
    yi = jnp.int32(0x5F3759DF) - (xi >> 1)
    y = lax.bitcast_convert_type(yi, jnp.float32)
    for _ in range(3):
        y = y * (1.5 - 0.5 * x * y * y)
    return y


@functools.partial(
    pl.kernel,
    out_type=[
        jax.ShapeDtypeStruct((K + 1, NC, N, DH), jnp.float32),
        jax.ShapeDtypeStruct((N, 16), jnp.float32),
    ],
    mesh=_mesh,
    scratch_types=[
        pltpu.VMEM((NCH, C), jnp.int32),
        pltpu.VMEM((NCH, C), jnp.int32),
        pltpu.VMEM((C, DH), jnp.float32),
        pltpu.VMEM((C, DH), jnp.float32),
        pltpu.VMEM((C, DH), jnp.float32),
        pltpu.VMEM((C, DH), jnp.float32),
        pltpu.VMEM((C, 16), jnp.float32),
        pltpu.SMEM((RPAD,), jnp.float32),
        pltpu.SMEM((RPAD,), jnp.float32),
        pltpu.VMEM_SHARED((N, DH), jnp.float32),
        pltpu.VMEM_SHARED((N, 16), jnp.float32),
        pltpu.SemaphoreType.DMA,
        pltpu.SemaphoreType.DMA,
        pltpu.SemaphoreType.DMA,
        pltpu.SemaphoreType.DMA,
        pltpu.SemaphoreType.DMA,
        pltpu.SemaphoreType.DMA,
        pltpu.SemaphoreType.DMA,
        pltpu.SemaphoreType.DMA,
    ],
    compiler_params=_sc_params,
)
def _multihop_kernel(fsplit_hbm, src_hbm, dst_hbm, gall_hbm, degs_hbm,
                     srcv, dstv, b0, b1, b2, b3, small, invds, normds,
                     acc, dacc,
                     sg0, sg1, sg2, sg3, ss0, ss1, ss2, ss3):
    c = lax.axis_index("c")
    s = lax.axis_index("s")
    bufs = (b0, b1, b2, b3)
    gsems = (sg0, sg1, sg2, sg3)
    ssems = (ss0, ss1, ss2, ss3)

    pltpu.sync_copy(src_hbm.at[s], srcv)
    pltpu.sync_copy(dst_hbm.at[s], dstv)

    def _zsmall(r, _):
        small[r, :] = jnp.zeros((16,), jnp.float32)
        return 0
    lax.fori_loop(0, C, _zsmall, 0)
    for q in range(NZ):
        pltpu.sync_copy(small, dacc.at[pl.ds(s * RT + q * C, C)])
    plsc.subcore_barrier()

    def _osmall(r, _):
        small[r, :] = jnp.ones((16,), jnp.float32)
        return 0
    lax.fori_loop(0, C, _osmall, 0)

    def _dchunk(j, _):
        pltpu.sync_copy(small, dacc.at[dstv.at[j]], add=True)
        return 0
    lax.fori_loop(0, NCH, _dchunk, 0)
    plsc.subcore_barrier()

    @pl.when(c == 0)
    def _():
        pltpu.sync_copy(dacc.at[pl.ds(s * DT, DT)],
                        degs_hbm.at[pl.ds(s * DT, DT)])

        @pl.when(s == NS - 1)
        def _():
            pltpu.sync_copy(dacc.at[pl.ds(NS * DT, DREM)],
                            degs_hbm.at[pl.ds(NS * DT, DREM)])

    for q in range(NZ):
        base = s * RT + q * C
        pltpu.sync_copy(dacc.at[pl.ds(base, C)], small)

        def _scal(r, _):
            dm = jnp.maximum(small[r, :], 1.0)
            iv = 1.0 / dm
            nr = _newton_rsqrt(dm)
            invds[q * C + r] = iv[0]
            normds[q * C + r] = nr[0]
            return 0
        lax.fori_loop(0, C, _scal, 0)

        pltpu.sync_copy(fsplit_hbm.at[c, pl.ds(base, C)], b0)

        def _g0row(r, _):
            nr = normds[q * C + r]
            for d4 in range(DH // 16):
                sl = pl.ds(d4 * 16, 16)
                b0[r, sl] = b0[r, sl] * nr
            return 0
        lax.fori_loop(0, C, _g0row, 0)
        pltpu.sync_copy(b0, gall_hbm.at[0, c, pl.ds(base, C)])
    plsc.subcore_barrier()

    def run_hop(k):
        gsrc = gall_hbm.at[k, c]

        def _zrow(r, _):
            for d4 in range(DH // 16):
                b3[r, pl.ds(d4 * 16, 16)] = jnp.zeros((16,), jnp.float32)
            return 0
        lax.fori_loop(0, C, _zrow, 0)
        for q in range(NZ):
            pltpu.sync_copy(b3, acc.at[pl.ds(s * RT + q * C, C)])
        plsc.subcore_barrier()

        def gstart(j, u):
            pltpu.async_copy(gsrc.at[srcv.at[j]], bufs[u], gsems[u])

        def gwait(j, u):
            pltpu.make_async_copy(gsrc.at[srcv.at[j]], bufs[u],
                                  gsems[u]).wait()

        def sstart(j, u):
            pltpu.async_copy(bufs[u], acc.at[dstv.at[j]], ssems[u],
                             add=True)

        def swait(j, u):
            pltpu.make_async_copy(bufs[u], acc.at[dstv.at[j]],
                                  ssems[u]).wait()

        gstart(0, 0)
        gstart(1, 1)

        def _group(i, _):
            for u in range(4):
                j = 4 * i + u
                gwait(j, u)
                sstart(j, u)
                if u >= 2:
                    swait(j - 2, (u - 2) % 4)
                else:
                    @pl.when(j >= 2)
                    def _():
                        swait(j - 2, (u - 2) % 4)
                if u < 2:
                    gstart(j + 2, (u + 2) % 4)
                else:
                    @pl.when(j + 2 < NCH)
                    def _():
                        gstart(j + 2, (u + 2) % 4)
            return 0

        lax.fori_loop(0, NCH // 4, _group, 0)
        swait(NCH - 2, (NCH - 2) % 4)
        swait(NCH - 1, (NCH - 1) % 4)
        plsc.subcore_barrier()

        for q in range(NZ):
            base = s * RT + q * C
            pltpu.sync_copy(acc.at[pl.ds(base, C)], b0)

            def _srow(r, _):
                iv = invds[q * C + r]
                for d4 in range(DH // 16):
                    sl = pl.ds(d4 * 16, 16)
                    b0[r, sl] = b0[r, sl] * iv
                return 0
            lax.fori_loop(0, C, _srow, 0)
            pltpu.sync_copy(b0, gall_hbm.at[k + 1, c, pl.ds(base, C)])
        plsc.subcore_barrier()

    def _hop(k, _):
        run_hop(k)
        return 0
    lax.fori_loop(0, K, _hop, 0)


_NB = 2000
_GRID = N // _NB


def _tc_final_body(gall_ref, degs_ref, s_ref, out_ref):
    sqd = jnp.sqrt(jnp.maximum(degs_ref[:, 0:1], 1.0))
    sv = s_ref[...]
    acc = jnp.zeros((_FB, D), jnp.float32)
    for k in range(K + 1):
        g = jnp.concatenate([gall_ref[k, 0], gall_ref[k, 1]], axis=-1)
        h = g * sqd
        t = jnp.dot(h, sv, preferred_element_type=jnp.float32)
        acc = acc + jax.nn.sigmoid(t) * h
    out_ref[...] = acc


_FB = 1000
_FGRID = N // _FB

_tc_final = pl.pallas_call(
    _tc_final_body,
    grid=(_FGRID,),
    in_specs=[
        pl.BlockSpec((K + 1, NC, _FB, DH), lambda i: (0, 0, i, 0)),
        pl.BlockSpec((_FB, 16), lambda i: (i, 0)),
        pl.BlockSpec((D, 1), lambda i: (0, 0)),
    ],
    out_specs=pl.BlockSpec((_FB, D), lambda i: (i, 0)),
    out_shape=jax.ShapeDtypeStruct((N, D), jnp.float32),
)


def kernel(features, edge_index, s):
    src = edge_index[0].reshape(NS, NCH, C)
    dst = edge_index[1].reshape(NS, NCH, C)
    fsplit = jnp.stack([features[:, :DH], features[:, DH:]])
    gall, degs = _multihop_kernel(fsplit, src, dst)
    return _tc_final(gall, degs, s)


# speedup vs baseline: 1.0348x; 1.0203x over previous
import functools

import jax
import jax.numpy as jnp
from jax import lax
from jax.experimental import pallas as pl
from jax.experimental.pallas import tpu as pltpu
from jax.experimental.pallas import tpu_sc as plsc

N = 10000
E = 320000
D = 128
K = 10

NC = 2
NS = 16
DH = D // NC

C = 125
ES = E // NS
NCH = ES // C

RT = N // NS
NZ = RT // C
DT = 624
DREM = N - NS * DT
RPAD = 640

_mesh = plsc.VectorSubcoreMesh(core_axis_name="c", subcore_axis_name="s")
_sc_params = pltpu.CompilerParams(use_tc_tiling_on_sc=False)


def _newton_rsqrt(x):
    xi = lax.bitcast_convert_type(x, jnp.int32)
    yi = jnp.int32(0x5F3759DF) - (xi >> 1)
    y = lax.bitcast_convert_type(yi, jnp.float32)
    for _ in range(3):
        y = y * (1.5 - 0.5 * x * y * y)
    return y


@functools.partial(
    pl.kernel,
    out_type=[
        jax.ShapeDtypeStruct((K + 1, NC, N, DH), jnp.float32),
        jax.ShapeDtypeStruct((N, 16), jnp.float32),
    ],
    mesh=_mesh,
    scratch_types=[
        pltpu.VMEM((NCH, C), jnp.int32),
        pltpu.VMEM((NCH, C), jnp.int32),
        pltpu.VMEM((C, DH), jnp.float32),
        pltpu.VMEM((C, DH), jnp.float32),
        pltpu.VMEM((C, DH), jnp.float32),
        pltpu.VMEM((C, DH), jnp.float32),
        pltpu.VMEM((C, 16), jnp.float32),
        pltpu.SMEM((RPAD,), jnp.float32),
        pltpu.SMEM((RPAD,), jnp.float32),
        pltpu.VMEM_SHARED((N, DH), jnp.float32),
        pltpu.VMEM_SHARED((N, 16), jnp.float32),
        pltpu.SemaphoreType.DMA,
        pltpu.SemaphoreType.DMA,
        pltpu.SemaphoreType.DMA,
        pltpu.SemaphoreType.DMA,
        pltpu.SemaphoreType.DMA,
        pltpu.SemaphoreType.DMA,
        pltpu.SemaphoreType.DMA,
        pltpu.SemaphoreType.DMA,
    ],
    compiler_params=_sc_params,
)
def _multihop_kernel(fsplit_hbm, src_hbm, dst_hbm, gall_hbm, degs_hbm,
                     srcv, dstv, b0, b1, b2, b3, small, invds, normds,
                     acc, dacc,
                     sg0, sg1, sg2, sg3, ss0, ss1, ss2, ss3):
    c = lax.axis_index("c")
    s = lax.axis_index("s")
    bufs = (b0, b1, b2, b3)
    gsems = (sg0, sg1, sg2, sg3)
    ssems = (ss0, ss1, ss2, ss3)

    pltpu.sync_copy(src_hbm.at[s], srcv)
    pltpu.sync_copy(dst_hbm.at[s], dstv)

    def _zsmall(r, _):
        small[r, :] = jnp.zeros((16,), jnp.float32)
        return 0
    lax.fori_loop(0, C, _zsmall, 0)
    for q in range(NZ):
        pltpu.sync_copy(small, dacc.at[pl.ds(s * RT + q * C, C)])
    plsc.subcore_barrier()

    def _osmall(r, _):
        small[r, :] = jnp.ones((16,), jnp.float32)
        return 0
    lax.fori_loop(0, C, _osmall, 0)

    def _dchunk(j, _):
        pltpu.sync_copy(small, dacc.at[dstv.at[j]], add=True)
        return 0
    lax.fori_loop(0, NCH, _dchunk, 0)
    plsc.subcore_barrier()

    @pl.when(c == 0)
    def _():
        pltpu.sync_copy(dacc.at[pl.ds(s * DT, DT)],
                        degs_hbm.at[pl.ds(s * DT, DT)])

        @pl.when(s == NS - 1)
        def _():
            pltpu.sync_copy(dacc.at[pl.ds(NS * DT, DREM)],
                            degs_hbm.at[pl.ds(NS * DT, DREM)])

    for q in range(NZ):
        base = s * RT + q * C
        pltpu.sync_copy(dacc.at[pl.ds(base, C)], small)

        def _scal(r, _):
            dm = jnp.maximum(small[r, :], 1.0)
            iv = 1.0 / dm
            nr = _newton_rsqrt(dm)
            invds[q * C + r] = iv[0]
            normds[q * C + r] = nr[0]
            return 0
        lax.fori_loop(0, C, _scal, 0)

        pltpu.sync_copy(fsplit_hbm.at[c, pl.ds(base, C)], b0)

        def _g0row(r, _):
            nr = normds[q * C + r]
            for d4 in range(DH // 16):
                sl = pl.ds(d4 * 16, 16)
                b0[r, sl] = b0[r, sl] * nr
            return 0
        lax.fori_loop(0, C, _g0row, 0)
        pltpu.sync_copy(b0, gall_hbm.at[0, c, pl.ds(base, C)])
    plsc.subcore_barrier()

    def run_hop(k):
        gsrc = gall_hbm.at[k, c]

        def _zrow(r, _):
            for d4 in range(DH // 16):
                b3[r, pl.ds(d4 * 16, 16)] = jnp.zeros((16,), jnp.float32)
            return 0
        lax.fori_loop(0, C, _zrow, 0)
        for q in range(NZ):
            pltpu.sync_copy(b3, acc.at[pl.ds(s * RT + q * C, C)])
        plsc.subcore_barrier()

        def gstart(j, u):
            pltpu.async_copy(gsrc.at[srcv.at[j]], bufs[u], gsems[u])

        def gwait(j, u):
            pltpu.make_async_copy(gsrc.at[srcv.at[j]], bufs[u],
                                  gsems[u]).wait()

        def sstart(j, u):
            pltpu.async_copy(bufs[u], acc.at[dstv.at[j]], ssems[u],
                             add=True)

        def swait(j, u):
            pltpu.make_async_copy(bufs[u], acc.at[dstv.at[j]],
                                  ssems[u]).wait()

        gstart(0, 0)
        gstart(1, 1)

        def _group(i, _):
            for u in range(4):
                j = 4 * i + u
                gwait(j, u)
                sstart(j, u)
                if u >= 2:
                    swait(j - 2, (u - 2) % 4)
                else:
                    @pl.when(j >= 2)
                    def _():
                        swait(j - 2, (u - 2) % 4)
                if u < 2:
                    gstart(j + 2, (u + 2) % 4)
                else:
                    @pl.when(j + 2 < NCH)
                    def _():
                        gstart(j + 2, (u + 2) % 4)
            return 0

        lax.fori_loop(0, NCH // 4, _group, 0)
        swait(NCH - 2, (NCH - 2) % 4)
        swait(NCH - 1, (NCH - 1) % 4)
        plsc.subcore_barrier()

        rbufs = (b0, b1, b2)

        def rd(q):
            return pltpu.make_async_copy(
                acc.at[pl.ds(s * RT + q * C, C)], rbufs[q % 3],
                gsems[q % 3])

        def wr(q):
            return pltpu.make_async_copy(
                rbufs[q % 3], gall_hbm.at[k + 1, c, pl.ds(s * RT + q * C, C)],
                ssems[q % 3])

        rd(0).start()
        rd(1).start()
        for q in range(NZ):
            bu = rbufs[q % 3]
            rd(q).wait()

            def _srow(r, _):
                iv = invds[q * C + r]
                for d4 in range(DH // 16):
                    sl = pl.ds(d4 * 16, 16)
                    bu[r, sl] = bu[r, sl] * iv
                return 0
            lax.fori_loop(0, C, _srow, 0)
            wr(q).start()
            if q + 2 < NZ:
                if q >= 1:
                    wr(q - 1).wait()
                rd(q + 2).start()
        wr(NZ - 3).wait()
        wr(NZ - 2).wait()
        wr(NZ - 1).wait()
        plsc.subcore_barrier()

    def _hop(k, _):
        run_hop(k)
        return 0
    lax.fori_loop(0, K, _hop, 0)


_NB = 2000
_GRID = N // _NB


def _tc_final_body(gall_ref, degs_ref, s_ref, out_ref):
    sqd = jnp.sqrt(jnp.maximum(degs_ref[:, 0:1], 1.0))
    sv = s_ref[...]
    acc = jnp.zeros((_FB, D), jnp.float32)
    for k in range(K + 1):
        g = jnp.concatenate([gall_ref[k, 0], gall_ref[k, 1]], axis=-1)
        h = g * sqd
        t = jnp.dot(h, sv, preferred_element_type=jnp.float32)
        acc = acc + jax.nn.sigmoid(t) * h
    out_ref[...] = acc


_FB = 1000
_FGRID = N // _FB

_tc_final = pl.pallas_call(
    _tc_final_body,
    grid=(_FGRID,),
    in_specs=[
        pl.BlockSpec((K + 1, NC, _FB, DH), lambda i: (0, 0, i, 0)),
        pl.BlockSpec((_FB, 16), lambda i: (i, 0)),
        pl.BlockSpec((D, 1), lambda i: (0, 0)),
    ],
    out_specs=pl.BlockSpec((_FB, D), lambda i: (i, 0)),
    out_shape=jax.ShapeDtypeStruct((N, D), jnp.float32),
)


def kernel(features, edge_index, s):
    src = edge_index[0].reshape(NS, NCH, C)
    dst = edge_index[1].reshape(NS, NCH, C)
    fsplit = jnp.stack([features[:, :DH], features[:, DH:]])
    gall, degs = _multihop_kernel(fsplit, src, dst)
    return _tc_final(gall, degs, s)


# speedup vs baseline: 1.0388x; 1.0039x over previous
import functools

import jax
import jax.numpy as jnp
from jax import lax
from jax.experimental import pallas as pl
from jax.experimental.pallas import tpu as pltpu
from jax.experimental.pallas import tpu_sc as plsc

N = 10000
E = 320000
D = 128
K = 10

NC = 2
NS = 16
DH = D // NC

C = 125
ES = E // NS
NCH = ES // C

RT = N // NS
NZ = RT // C
DT = 624
DREM = N - NS * DT
RPAD = 640

_mesh = plsc.VectorSubcoreMesh(core_axis_name="c", subcore_axis_name="s")
_sc_params = pltpu.CompilerParams(use_tc_tiling_on_sc=False)


def _newton_rsqrt(x):
    xi = lax.bitcast_convert_type(x, jnp.int32)
    yi = jnp.int32(0x5F3759DF) - (xi >> 1)
    y = lax.bitcast_convert_type(yi, jnp.float32)
    for _ in range(3):
        y = y * (1.5 - 0.5 * x * y * y)
    return y


@functools.partial(
    pl.kernel,
    out_type=[
        jax.ShapeDtypeStruct((K + 1, NC, N, DH), jnp.float32),
        jax.ShapeDtypeStruct((N, 16), jnp.float32),
    ],
    mesh=_mesh,
    scratch_types=[
        pltpu.VMEM((NCH, C), jnp.int32),
        pltpu.VMEM((NCH, C), jnp.int32),
        pltpu.VMEM((C, DH), jnp.float32),
        pltpu.VMEM((C, DH), jnp.float32),
        pltpu.VMEM((C, DH), jnp.float32),
        pltpu.VMEM((C, DH), jnp.float32),
        pltpu.VMEM((C, 16), jnp.float32),
        pltpu.SMEM((RPAD,), jnp.float32),
        pltpu.SMEM((RPAD,), jnp.float32),
        pltpu.VMEM_SHARED((N, DH), jnp.float32),
        pltpu.VMEM_SHARED((N, 16), jnp.float32),
        pltpu.SemaphoreType.DMA,
        pltpu.SemaphoreType.DMA,
        pltpu.SemaphoreType.DMA,
        pltpu.SemaphoreType.DMA,
        pltpu.SemaphoreType.DMA,
        pltpu.SemaphoreType.DMA,
        pltpu.SemaphoreType.DMA,
        pltpu.SemaphoreType.DMA,
    ],
    compiler_params=_sc_params,
)
def _multihop_kernel(fsplit_hbm, src_hbm, dst_hbm, gall_hbm, degs_hbm,
                     srcv, dstv, b0, b1, b2, b3, small, invds, normds,
                     acc, dacc,
                     sg0, sg1, sg2, sg3, ss0, ss1, ss2, ss3):
    c = lax.axis_index("c")
    s = lax.axis_index("s")
    bufs = (b0, b1, b2, b3)
    gsems = (sg0, sg1, sg2, sg3)
    ssems = (ss0, ss1, ss2, ss3)

    pltpu.sync_copy(src_hbm.at[s], srcv)
    pltpu.sync_copy(dst_hbm.at[s], dstv)

    def _zsmall(r, _):
        small[r, :] = jnp.zeros((16,), jnp.float32)
        return 0
    lax.fori_loop(0, C, _zsmall, 0)
    for q in range(NZ):
        pltpu.sync_copy(small, dacc.at[pl.ds(s * RT + q * C, C)])
    plsc.subcore_barrier()

    def _osmall(r, _):
        small[r, :] = jnp.ones((16,), jnp.float32)
        return 0
    lax.fori_loop(0, C, _osmall, 0)

    def _dchunk(j, _):
        pltpu.async_copy(small, dacc.at[dstv.at[j]], sg0, add=True)
        return 0
    lax.fori_loop(0, NCH, _dchunk, 0)

    def _ddrain(j, _):
        pltpu.make_async_copy(small, dacc.at[dstv.at[j]], sg0).wait()
        return 0
    lax.fori_loop(0, NCH, _ddrain, 0)
    plsc.subcore_barrier()

    @pl.when(c == 0)
    def _():
        pltpu.sync_copy(dacc.at[pl.ds(s * DT, DT)],
                        degs_hbm.at[pl.ds(s * DT, DT)])

        @pl.when(s == NS - 1)
        def _():
            pltpu.sync_copy(dacc.at[pl.ds(NS * DT, DREM)],
                            degs_hbm.at[pl.ds(NS * DT, DREM)])

    for q in range(NZ):
        base = s * RT + q * C
        pltpu.sync_copy(dacc.at[pl.ds(base, C)], small)

        def _scal(r, _):
            dm = jnp.maximum(small[r, :], 1.0)
            iv = 1.0 / dm
            nr = _newton_rsqrt(dm)
            invds[q * C + r] = iv[0]
            normds[q * C + r] = nr[0]
            return 0
        lax.fori_loop(0, C, _scal, 0)

        pltpu.sync_copy(fsplit_hbm.at[c, pl.ds(base, C)], b0)

        def _g0row(r, _):
            nr = normds[q * C + r]
            for d4 in range(DH // 16):
                sl = pl.ds(d4 * 16, 16)
                b0[r, sl] = b0[r, sl] * nr
            return 0
        lax.fori_loop(0, C, _g0row, 0)
        pltpu.sync_copy(b0, gall_hbm.at[0, c, pl.ds(base, C)])
    plsc.subcore_barrier()

    def run_hop(k):
        gsrc = gall_hbm.at[k, c]

        def _zrow(r, _):
            for d4 in range(DH // 16):
                b3[r, pl.ds(d4 * 16, 16)] = jnp.zeros((16,), jnp.float32)
            return 0
        lax.fori_loop(0, C, _zrow, 0)
        for q in range(NZ):
            pltpu.sync_copy(b3, acc.at[pl.ds(s * RT + q * C, C)])
        plsc.subcore_barrier()

        def gstart(j, u):
            pltpu.async_copy(gsrc.at[srcv.at[j]], bufs[u], gsems[u])

        def gwait(j, u):
            pltpu.make_async_copy(gsrc.at[srcv.at[j]], bufs[u],
                                  gsems[u]).wait()

        def sstart(j, u):
            pltpu.async_copy(bufs[u], acc.at[dstv.at[j]], ssems[u],
                             add=True)

        def swait(j, u):
            pltpu.make_async_copy(bufs[u], acc.at[dstv.at[j]],
                                  ssems[u]).wait()

        gstart(0, 0)
        gstart(1, 1)

        def _group(i, _):
            for u in range(4):
                j = 4 * i + u
                gwait(j, u)
                sstart(j, u)
                if u >= 2:
                    swait(j - 2, (u - 2) % 4)
                else:
                    @pl.when(j >= 2)
                    def _():
                        swait(j - 2, (u - 2) % 4)
                if u < 2:
                    gstart(j + 2, (u + 2) % 4)
                else:
                    @pl.when(j + 2 < NCH)
                    def _():
                        gstart(j + 2, (u + 2) % 4)
            return 0

        lax.fori_loop(0, NCH // 4, _group, 0)
        swait(NCH - 2, (NCH - 2) % 4)
        swait(NCH - 1, (NCH - 1) % 4)
        plsc.subcore_barrier()

        rbufs = (b0, b1, b2)

        def rd(q):
            return pltpu.make_async_copy(
                acc.at[pl.ds(s * RT + q * C, C)], rbufs[q % 3],
                gsems[q % 3])

        def wr(q):
            return pltpu.make_async_copy(
                rbufs[q % 3], gall_hbm.at[k + 1, c, pl.ds(s * RT + q * C, C)],
                ssems[q % 3])

        rd(0).start()
        rd(1).start()
        for q in range(NZ):
            bu = rbufs[q % 3]
            rd(q).wait()

            def _srow(r, _):
                iv = invds[q * C + r]
                for d4 in range(DH // 16):
                    sl = pl.ds(d4 * 16, 16)
                    bu[r, sl] = bu[r, sl] * iv
                return 0
            lax.fori_loop(0, C, _srow, 0)
            wr(q).start()
            if q + 2 < NZ:
                if q >= 1:
                    wr(q - 1).wait()
                rd(q + 2).start()
        wr(NZ - 3).wait()
        wr(NZ - 2).wait()
        wr(NZ - 1).wait()
        plsc.subcore_barrier()

    def _hop(k, _):
        run_hop(k)
        return 0
    lax.fori_loop(0, K, _hop, 0)


_NB = 2000
_GRID = N // _NB


def _tc_final_body(gall_ref, degs_ref, s_ref, out_ref):
    sqd = jnp.sqrt(jnp.maximum(degs_ref[:, 0:1], 1.0))
    sv = s_ref[...]
    acc = jnp.zeros((_FB, D), jnp.float32)
    for k in range(K + 1):
        g = jnp.concatenate([gall_ref[k, 0], gall_ref[k, 1]], axis=-1)
        h = g * sqd
        t = jnp.dot(h, sv, preferred_element_type=jnp.float32)
        acc = acc + jax.nn.sigmoid(t) * h
    out_ref[...] = acc


_FB = 1000
_FGRID = N // _FB

_tc_final = pl.pallas_call(
    _tc_final_body,
    grid=(_FGRID,),
    in_specs=[
        pl.BlockSpec((K + 1, NC, _FB, DH), lambda i: (0, 0, i, 0)),
        pl.BlockSpec((_FB, 16), lambda i: (i, 0)),
        pl.BlockSpec((D, 1), lambda i: (0, 0)),
    ],
    out_specs=pl.BlockSpec((_FB, D), lambda i: (i, 0)),
    out_shape=jax.ShapeDtypeStruct((N, D), jnp.float32),
)


def kernel(features, edge_index, s):
    src = edge_index[0].reshape(NS, NCH, C)
    dst = edge_index[1].reshape(NS, NCH, C)
    fsplit = jnp.stack([features[:, :DH], features[:, DH:]])
    gall, degs = _multihop_kernel(fsplit, src, dst)
    return _tc_final(gall, degs, s)
